# baseline (speedup 1.0000x reference)
import numpy as np

import jax
import jax.numpy as jnp
from jax.experimental import pallas as pl
from jax.experimental.pallas import tpu as pltpu

B_TILE = 128
N_CLASSES = 10


def _fused_kernel(x_ref, w1p_ref, b1p_ref, w2p_ref, b2p_ref,
                  wf1p_ref, bf1_ref, wf2_ref, bf2_ref, o_ref):
    bt = x_ref.shape[0]
    xb = x_ref[...].astype(jnp.bfloat16)
    xb = jnp.concatenate([xb, xb[:, 0:4, :]], axis=1)

    xcat = jnp.concatenate([xb[:, 0:16, :], xb[:, 1:17, :], xb[:, 2:18, :]],
                           axis=2)

    cand = []
    for di in range(2):
        lhs = xcat[:, :, 28 * di:28 * di + 140].reshape(bt * 16, 140)
        cand.append(jnp.dot(lhs, w1p_ref[...],
                            preferred_element_type=jnp.float32))
    t = jnp.maximum(cand[0], cand[1])
    m = jnp.maximum(t[:, 0:120], t[:, 128:248]).reshape(bt, 16, 120)
    h1 = jnp.maximum(m + b1p_ref[...], 0.0).astype(jnp.bfloat16)

    acc2 = None
    for kg in range(3):
        lhs = jnp.concatenate(
            [h1[:, 2 * kg:2 * kg + 8, :], h1[:, 2 * kg + 1:2 * kg + 9, :]],
            axis=2).reshape(bt * 8, 240)
        d = jnp.dot(lhs, w2p_ref[kg],
                    preferred_element_type=jnp.float32)
        acc2 = d if acc2 is None else acc2 + d
    zc = jnp.maximum(acc2[:, 0:80], acc2[:, 128:208]).reshape(bt, 8, 80)
    zm = jnp.maximum(zc[:, 0:7, :], zc[:, 1:8, :])
    h2 = jnp.maximum(zm + b2p_ref[...], 0.0).astype(jnp.bfloat16)

    ha = None
    for i2 in range(4):
        d = jnp.dot(h2[:, 2 * i2, :], wf1p_ref[i2],
                    preferred_element_type=jnp.float32)
        ha = d if ha is None else ha + d
    h = jnp.maximum(ha + bf1_ref[...], 0.0).astype(jnp.bfloat16)
    y = jnp.dot(h, wf2_ref[...],
                preferred_element_type=jnp.float32) + bf2_ref[...]

    lane = jax.lax.broadcasted_iota(jnp.int32, (1, 128), 1)
    y = jnp.where(lane < N_CLASSES, y, -1e30)
    mx = jnp.max(y, axis=-1, keepdims=True)
    lse = jnp.log(jnp.sum(jnp.exp(y - mx), axis=-1, keepdims=True)) + mx
    o_ref[...] = y - lse


_E1 = np.zeros((2, 28, 12, 5), np.float32)
for _h in range(2):
    for _j in range(12):
        for _kj in range(5):
            _E1[_h, 2 * _j + _h + _kj, _j, _kj] = 1.0
_E2 = np.zeros((2, 12, 4, 5), np.float32)
for _h in range(2):
    for _j in range(4):
        for _kj in range(5):
            _E2[_h, 2 * _j + _h + _kj, _j, _kj] = 1.0


def _prep(w1, b1, w2, b2, wf1):
    k1 = w1[:25, :10].astype(jnp.float32).reshape(5, 5, 10)
    w1p = jnp.einsum('hNjk,ikc->iNhjc', jnp.asarray(_E1), k1)
    w1p = jnp.pad(w1p.reshape(5, 28, 2, 120), ((0, 0),) * 3 + ((0, 8),))
    w1p = w1p.reshape(140, 256).astype(jnp.bfloat16)
    b1p = jnp.tile(b1[:, :10], (1, 12))

    k2 = w2.reshape(5, 5, 128, 128)[:, :, :10, :20].astype(jnp.float32)
    w2f = jnp.einsum('hNjk,ikcd->iNchjd', jnp.asarray(_E2), k2)
    w2f = jnp.pad(w2f.reshape(5, 120, 2, 80), ((0, 0),) * 3 + ((0, 48),))
    w2f = w2f.reshape(5, 120, 256)
    w2f = jnp.concatenate([w2f, jnp.zeros((1, 120, 256))], axis=0)
    w2p = w2f.reshape(3, 240, 256).astype(jnp.bfloat16)
    b2p = jnp.tile(b2[:, :20], (1, 4))

    wf1p = wf1.reshape(4, 4, 128, 128)[:, :, :20, :].reshape(4, 80, 128)
    return w1p, b1p, w2p, b2p, wf1p


@jax.jit
def kernel(x, w1, b1, w2, b2, wf1, bf1, wf2, bf2):
    B = x.shape[0]
    w1p, b1p, w2p, b2p, wf1p = _prep(w1, b1, w2, b2, wf1)
    xp = x.reshape(B, 14, 56)

    out = pl.pallas_call(
        _fused_kernel,
        out_shape=jax.ShapeDtypeStruct((B, 128), jnp.float32),
        grid=(B // B_TILE,),
        in_specs=[
            pl.BlockSpec((B_TILE, 14, 56), lambda b: (b, 0, 0)),
            pl.BlockSpec((140, 256), lambda b: (0, 0)),
            pl.BlockSpec((1, 120), lambda b: (0, 0)),
            pl.BlockSpec((3, 240, 256), lambda b: (0, 0, 0)),
            pl.BlockSpec((1, 80), lambda b: (0, 0)),
            pl.BlockSpec((4, 80, 128), lambda b: (0, 0, 0)),
            pl.BlockSpec((1, 128), lambda b: (0, 0)),
            pl.BlockSpec((128, 128), lambda b: (0, 0)),
            pl.BlockSpec((1, 128), lambda b: (0, 0)),
        ],
        out_specs=pl.BlockSpec((B_TILE, 128), lambda b: (b, 0)),
        compiler_params=pltpu.CompilerParams(
            dimension_semantics=("parallel",),
            vmem_limit_bytes=64 * 1024 * 1024),
    )(xp, w1p, b1p, w2p, b2p, wf1p, bf1, wf2, bf2)

    return out[:B, :N_CLASSES]


# speedup vs baseline: 1.0946x; 1.0946x over previous
import numpy as np

import jax
import jax.numpy as jnp
from jax.experimental import pallas as pl
from jax.experimental.pallas import tpu as pltpu

B_TILE = 256
N_CLASSES = 10


def _fused_kernel(x_ref, w1p_ref, b1p_ref, w2p_ref, b2p_ref,
                  wf1p_ref, bf1_ref, wf2_ref, bf2_ref, o_ref):
    bt = x_ref.shape[0]
    xb = x_ref[...].astype(jnp.bfloat16)
    xb = jnp.concatenate([xb, xb[:, 0:4, :]], axis=1)

    xcat = jnp.concatenate([xb[:, 0:16, :], xb[:, 1:17, :], xb[:, 2:18, :]],
                           axis=2)

    cand = []
    for di in range(2):
        lhs = xcat[:, :, 28 * di:28 * di + 140].reshape(bt * 16, 140)
        cand.append(jnp.dot(lhs, w1p_ref[...],
                            preferred_element_type=jnp.float32))
    t = jnp.maximum(cand[0], cand[1])
    m = jnp.maximum(t[:, 0:120], t[:, 128:248]).reshape(bt, 16, 120)
    h1 = jnp.maximum(m + b1p_ref[...], 0.0).astype(jnp.bfloat16)

    acc2 = None
    for kg in range(3):
        lhs = jnp.concatenate(
            [h1[:, 2 * kg:2 * kg + 8, :], h1[:, 2 * kg + 1:2 * kg + 9, :]],
            axis=2).reshape(bt * 8, 240)
        d = jnp.dot(lhs, w2p_ref[kg],
                    preferred_element_type=jnp.float32)
        acc2 = d if acc2 is None else acc2 + d
    zc = jnp.maximum(acc2[:, 0:80], acc2[:, 128:208]).reshape(bt, 8, 80)
    zm = jnp.maximum(zc[:, 0:7, :], zc[:, 1:8, :])
    h2 = jnp.maximum(zm + b2p_ref[...], 0.0).astype(jnp.bfloat16)

    ha = None
    for i2 in range(4):
        d = jnp.dot(h2[:, 2 * i2, :], wf1p_ref[i2],
                    preferred_element_type=jnp.float32)
        ha = d if ha is None else ha + d
    h = jnp.maximum(ha + bf1_ref[...], 0.0).astype(jnp.bfloat16)
    y = jnp.dot(h, wf2_ref[...],
                preferred_element_type=jnp.float32) + bf2_ref[...]

    lane = jax.lax.broadcasted_iota(jnp.int32, (1, 128), 1)
    y = jnp.where(lane < N_CLASSES, y, -1e30)
    mx = jnp.max(y, axis=-1, keepdims=True)
    lse = jnp.log(jnp.sum(jnp.exp(y - mx), axis=-1, keepdims=True)) + mx
    o_ref[...] = y - lse


_E1 = np.zeros((2, 28, 12, 5), np.float32)
for _h in range(2):
    for _j in range(12):
        for _kj in range(5):
            _E1[_h, 2 * _j + _h + _kj, _j, _kj] = 1.0
_E2 = np.zeros((2, 12, 4, 5), np.float32)
for _h in range(2):
    for _j in range(4):
        for _kj in range(5):
            _E2[_h, 2 * _j + _h + _kj, _j, _kj] = 1.0


def _prep(w1, b1, w2, b2, wf1):
    k1 = w1[:25, :10].astype(jnp.float32).reshape(5, 5, 10)
    w1p = jnp.einsum('hNjk,ikc->iNhjc', jnp.asarray(_E1), k1)
    w1p = jnp.pad(w1p.reshape(5, 28, 2, 120), ((0, 0),) * 3 + ((0, 8),))
    w1p = w1p.reshape(140, 256).astype(jnp.bfloat16)
    b1p = jnp.tile(b1[:, :10], (1, 12))

    k2 = w2.reshape(5, 5, 128, 128)[:, :, :10, :20].astype(jnp.float32)
    w2f = jnp.einsum('hNjk,ikcd->iNchjd', jnp.asarray(_E2), k2)
    w2f = jnp.pad(w2f.reshape(5, 120, 2, 80), ((0, 0),) * 3 + ((0, 48),))
    w2f = w2f.reshape(5, 120, 256)
    w2f = jnp.concatenate([w2f, jnp.zeros((1, 120, 256))], axis=0)
    w2p = w2f.reshape(3, 240, 256).astype(jnp.bfloat16)
    b2p = jnp.tile(b2[:, :20], (1, 4))

    wf1p = wf1.reshape(4, 4, 128, 128)[:, :, :20, :].reshape(4, 80, 128)
    return w1p, b1p, w2p, b2p, wf1p


@jax.jit
def kernel(x, w1, b1, w2, b2, wf1, bf1, wf2, bf2):
    B = x.shape[0]
    w1p, b1p, w2p, b2p, wf1p = _prep(w1, b1, w2, b2, wf1)
    xp = x.reshape(B, 14, 56)

    out = pl.pallas_call(
        _fused_kernel,
        out_shape=jax.ShapeDtypeStruct((B, 128), jnp.float32),
        grid=(B // B_TILE,),
        in_specs=[
            pl.BlockSpec((B_TILE, 14, 56), lambda b: (b, 0, 0)),
            pl.BlockSpec((140, 256), lambda b: (0, 0)),
            pl.BlockSpec((1, 120), lambda b: (0, 0)),
            pl.BlockSpec((3, 240, 256), lambda b: (0, 0, 0)),
            pl.BlockSpec((1, 80), lambda b: (0, 0)),
            pl.BlockSpec((4, 80, 128), lambda b: (0, 0, 0)),
            pl.BlockSpec((1, 128), lambda b: (0, 0)),
            pl.BlockSpec((128, 128), lambda b: (0, 0)),
            pl.BlockSpec((1, 128), lambda b: (0, 0)),
        ],
        out_specs=pl.BlockSpec((B_TILE, 128), lambda b: (b, 0)),
        compiler_params=pltpu.CompilerParams(
            dimension_semantics=("parallel",),
            vmem_limit_bytes=64 * 1024 * 1024),
    )(xp, w1p, b1p, w2p, b2p, wf1p, bf1, wf2, bf2)

    return out[:B, :N_CLASSES]


# speedup vs baseline: 1.1512x; 1.0517x over previous
import numpy as np

import jax
import jax.numpy as jnp
from jax.experimental import pallas as pl
from jax.experimental.pallas import tpu as pltpu

B_TILE = 512
N_CLASSES = 10


def _fused_kernel(x_ref, w1p_ref, b1p_ref, w2p_ref, b2p_ref,
                  wf1p_ref, bf1_ref, wf2_ref, bf2_ref, o_ref):
    bt = x_ref.shape[0]
    xb = x_ref[...].astype(jnp.bfloat16)
    xb = jnp.concatenate([xb, xb[:, 0:4, :]], axis=1)

    xcat = jnp.concatenate([xb[:, 0:16, :], xb[:, 1:17, :], xb[:, 2:18, :]],
                           axis=2)

    cand = []
    for di in range(2):
        lhs = xcat[:, :, 28 * di:28 * di + 140].reshape(bt * 16, 140)
        cand.append(jnp.dot(lhs, w1p_ref[...],
                            preferred_element_type=jnp.float32))
    t = jnp.maximum(cand[0], cand[1])
    m = jnp.maximum(t[:, 0:120], t[:, 128:248]).reshape(bt, 16, 120)
    h1 = jnp.maximum(m + b1p_ref[...], 0.0).astype(jnp.bfloat16)

    acc2 = None
    for kg in range(3):
        lhs = jnp.concatenate(
            [h1[:, 2 * kg:2 * kg + 8, :], h1[:, 2 * kg + 1:2 * kg + 9, :]],
            axis=2).reshape(bt * 8, 240)
        d = jnp.dot(lhs, w2p_ref[kg],
                    preferred_element_type=jnp.float32)
        acc2 = d if acc2 is None else acc2 + d
    zc = jnp.maximum(acc2[:, 0:80], acc2[:, 128:208]).reshape(bt, 8, 80)
    zm = jnp.maximum(zc[:, 0:7, :], zc[:, 1:8, :])
    h2 = jnp.maximum(zm + b2p_ref[...], 0.0).astype(jnp.bfloat16)

    ha = None
    for i2 in range(4):
        d = jnp.dot(h2[:, 2 * i2, :], wf1p_ref[i2],
                    preferred_element_type=jnp.float32)
        ha = d if ha is None else ha + d
    h = jnp.maximum(ha + bf1_ref[...], 0.0).astype(jnp.bfloat16)
    y = jnp.dot(h, wf2_ref[...],
                preferred_element_type=jnp.float32) + bf2_ref[...]

    lane = jax.lax.broadcasted_iota(jnp.int32, (1, 128), 1)
    y = jnp.where(lane < N_CLASSES, y, -1e30)
    mx = jnp.max(y, axis=-1, keepdims=True)
    lse = jnp.log(jnp.sum(jnp.exp(y - mx), axis=-1, keepdims=True)) + mx
    o_ref[...] = y - lse


_E1 = np.zeros((2, 28, 12, 5), np.float32)
for _h in range(2):
    for _j in range(12):
        for _kj in range(5):
            _E1[_h, 2 * _j + _h + _kj, _j, _kj] = 1.0
_E2 = np.zeros((2, 12, 4, 5), np.float32)
for _h in range(2):
    for _j in range(4):
        for _kj in range(5):
            _E2[_h, 2 * _j + _h + _kj, _j, _kj] = 1.0


def _prep(w1, b1, w2, b2, wf1):
    k1 = w1[:25, :10].astype(jnp.float32).reshape(5, 5, 10)
    w1p = jnp.einsum('hNjk,ikc->iNhjc', jnp.asarray(_E1), k1)
    w1p = jnp.pad(w1p.reshape(5, 28, 2, 120), ((0, 0),) * 3 + ((0, 8),))
    w1p = w1p.reshape(140, 256).astype(jnp.bfloat16)
    b1p = jnp.tile(b1[:, :10], (1, 12))

    k2 = w2.reshape(5, 5, 128, 128)[:, :, :10, :20].astype(jnp.float32)
    w2f = jnp.einsum('hNjk,ikcd->iNchjd', jnp.asarray(_E2), k2)
    w2f = jnp.pad(w2f.reshape(5, 120, 2, 80), ((0, 0),) * 3 + ((0, 48),))
    w2f = w2f.reshape(5, 120, 256)
    w2f = jnp.concatenate([w2f, jnp.zeros((1, 120, 256))], axis=0)
    w2p = w2f.reshape(3, 240, 256).astype(jnp.bfloat16)
    b2p = jnp.tile(b2[:, :20], (1, 4))

    wf1p = wf1.reshape(4, 4, 128, 128)[:, :, :20, :].reshape(4, 80, 128)
    return w1p, b1p, w2p, b2p, wf1p


@jax.jit
def kernel(x, w1, b1, w2, b2, wf1, bf1, wf2, bf2):
    B = x.shape[0]
    w1p, b1p, w2p, b2p, wf1p = _prep(w1, b1, w2, b2, wf1)
    xp = x.reshape(B, 14, 56)

    out = pl.pallas_call(
        _fused_kernel,
        out_shape=jax.ShapeDtypeStruct((B, 128), jnp.float32),
        grid=(B // B_TILE,),
        in_specs=[
            pl.BlockSpec((B_TILE, 14, 56), lambda b: (b, 0, 0)),
            pl.BlockSpec((140, 256), lambda b: (0, 0)),
            pl.BlockSpec((1, 120), lambda b: (0, 0)),
            pl.BlockSpec((3, 240, 256), lambda b: (0, 0, 0)),
            pl.BlockSpec((1, 80), lambda b: (0, 0)),
            pl.BlockSpec((4, 80, 128), lambda b: (0, 0, 0)),
            pl.BlockSpec((1, 128), lambda b: (0, 0)),
            pl.BlockSpec((128, 128), lambda b: (0, 0)),
            pl.BlockSpec((1, 128), lambda b: (0, 0)),
        ],
        out_specs=pl.BlockSpec((B_TILE, 128), lambda b: (b, 0)),
        compiler_params=pltpu.CompilerParams(
            dimension_semantics=("parallel",),
            vmem_limit_bytes=64 * 1024 * 1024),
    )(xp, w1p, b1p, w2p, b2p, wf1p, bf1, wf2, bf2)

    return out[:B, :N_CLASSES]


# speedup vs baseline: 1.1765x; 1.0220x over previous
import numpy as np

import jax
import jax.numpy as jnp
from jax.experimental import pallas as pl
from jax.experimental.pallas import tpu as pltpu

B_TILE = 1024
N_CLASSES = 10


def _fused_kernel(x_ref, w1p_ref, b1p_ref, w2p_ref, b2p_ref,
                  wf1p_ref, bf1_ref, wf2_ref, bf2_ref, o_ref):
    bt = x_ref.shape[0]
    xb = x_ref[...].astype(jnp.bfloat16)
    xb = jnp.concatenate([xb, xb[:, 0:4, :]], axis=1)

    xcat = jnp.concatenate([xb[:, 0:16, :], xb[:, 1:17, :], xb[:, 2:18, :]],
                           axis=2)

    cand = []
    for di in range(2):
        lhs = xcat[:, :, 28 * di:28 * di + 140].reshape(bt * 16, 140)
        cand.append(jnp.dot(lhs, w1p_ref[...],
                            preferred_element_type=jnp.float32))
    t = jnp.maximum(cand[0], cand[1])
    m = jnp.maximum(t[:, 0:120], t[:, 128:248]).reshape(bt, 16, 120)
    h1 = jnp.maximum(m + b1p_ref[...], 0.0).astype(jnp.bfloat16)

    acc2 = None
    for kg in range(3):
        lhs = jnp.concatenate(
            [h1[:, 2 * kg:2 * kg + 8, :], h1[:, 2 * kg + 1:2 * kg + 9, :]],
            axis=2).reshape(bt * 8, 240)
        d = jnp.dot(lhs, w2p_ref[kg],
                    preferred_element_type=jnp.float32)
        acc2 = d if acc2 is None else acc2 + d
    zc = jnp.maximum(acc2[:, 0:80], acc2[:, 128:208]).reshape(bt, 8, 80)
    zm = jnp.maximum(zc[:, 0:7, :], zc[:, 1:8, :])
    h2 = jnp.maximum(zm + b2p_ref[...], 0.0).astype(jnp.bfloat16)

    ha = None
    for i2 in range(4):
        d = jnp.dot(h2[:, 2 * i2, :], wf1p_ref[i2],
                    preferred_element_type=jnp.float32)
        ha = d if ha is None else ha + d
    h = jnp.maximum(ha + bf1_ref[...], 0.0).astype(jnp.bfloat16)
    y = jnp.dot(h, wf2_ref[...],
                preferred_element_type=jnp.float32) + bf2_ref[...]

    lane = jax.lax.broadcasted_iota(jnp.int32, (1, 128), 1)
    y = jnp.where(lane < N_CLASSES, y, -1e30)
    mx = jnp.max(y, axis=-1, keepdims=True)
    lse = jnp.log(jnp.sum(jnp.exp(y - mx), axis=-1, keepdims=True)) + mx
    o_ref[...] = y - lse


_E1 = np.zeros((2, 28, 12, 5), np.float32)
for _h in range(2):
    for _j in range(12):
        for _kj in range(5):
            _E1[_h, 2 * _j + _h + _kj, _j, _kj] = 1.0
_E2 = np.zeros((2, 12, 4, 5), np.float32)
for _h in range(2):
    for _j in range(4):
        for _kj in range(5):
            _E2[_h, 2 * _j + _h + _kj, _j, _kj] = 1.0


def _prep(w1, b1, w2, b2, wf1):
    k1 = w1[:25, :10].astype(jnp.float32).reshape(5, 5, 10)
    w1p = jnp.einsum('hNjk,ikc->iNhjc', jnp.asarray(_E1), k1)
    w1p = jnp.pad(w1p.reshape(5, 28, 2, 120), ((0, 0),) * 3 + ((0, 8),))
    w1p = w1p.reshape(140, 256).astype(jnp.bfloat16)
    b1p = jnp.tile(b1[:, :10], (1, 12))

    k2 = w2.reshape(5, 5, 128, 128)[:, :, :10, :20].astype(jnp.float32)
    w2f = jnp.einsum('hNjk,ikcd->iNchjd', jnp.asarray(_E2), k2)
    w2f = jnp.pad(w2f.reshape(5, 120, 2, 80), ((0, 0),) * 3 + ((0, 48),))
    w2f = w2f.reshape(5, 120, 256)
    w2f = jnp.concatenate([w2f, jnp.zeros((1, 120, 256))], axis=0)
    w2p = w2f.reshape(3, 240, 256).astype(jnp.bfloat16)
    b2p = jnp.tile(b2[:, :20], (1, 4))

    wf1p = wf1.reshape(4, 4, 128, 128)[:, :, :20, :].reshape(4, 80, 128)
    return w1p, b1p, w2p, b2p, wf1p


@jax.jit
def kernel(x, w1, b1, w2, b2, wf1, bf1, wf2, bf2):
    B = x.shape[0]
    w1p, b1p, w2p, b2p, wf1p = _prep(w1, b1, w2, b2, wf1)
    xp = x.reshape(B, 14, 56)

    out = pl.pallas_call(
        _fused_kernel,
        out_shape=jax.ShapeDtypeStruct((B, 128), jnp.float32),
        grid=(B // B_TILE,),
        in_specs=[
            pl.BlockSpec((B_TILE, 14, 56), lambda b: (b, 0, 0)),
            pl.BlockSpec((140, 256), lambda b: (0, 0)),
            pl.BlockSpec((1, 120), lambda b: (0, 0)),
            pl.BlockSpec((3, 240, 256), lambda b: (0, 0, 0)),
            pl.BlockSpec((1, 80), lambda b: (0, 0)),
            pl.BlockSpec((4, 80, 128), lambda b: (0, 0, 0)),
            pl.BlockSpec((1, 128), lambda b: (0, 0)),
            pl.BlockSpec((128, 128), lambda b: (0, 0)),
            pl.BlockSpec((1, 128), lambda b: (0, 0)),
        ],
        out_specs=pl.BlockSpec((B_TILE, 128), lambda b: (b, 0)),
        compiler_params=pltpu.CompilerParams(
            dimension_semantics=("parallel",),
            vmem_limit_bytes=64 * 1024 * 1024),
    )(xp, w1p, b1p, w2p, b2p, wf1p, bf1, wf2, bf2)

    return out[:B, :N_CLASSES]
